# baseline (speedup 1.0000x reference)
import functools

import jax
import jax.numpy as jnp
from jax import lax
from jax.experimental import pallas as pl
from jax.experimental.pallas import tpu as pltpu
from jax.experimental.pallas import tpu_sc as plsc

N_NODES = 10000
D = 128
NC = 2
NS = 16
NW = NC * NS
CHUNK = 128
ECHUNK = 128
NBUF = 2
N_PAD = 10240
RPT = N_PAD // NS
G = RPT // CHUNK
GA = RPT // ECHUNK


def _mesh():
    return plsc.VectorSubcoreMesh(
        core_axis_name="c", subcore_axis_name="s", num_cores=NC, num_subcores=NS
    )


def _fill_slice_indices(idxv, s, chunk, groups):
    iota16 = lax.iota(jnp.int32, 16)
    per_row = chunk // 16

    def fill(k, _):
        g = k // per_row
        t = k % per_row
        idxv[g, pl.ds(t * 16, 16)] = s * RPT + g * chunk + t * 16 + iota16
        return _

    lax.fori_loop(0, groups * per_row, fill, None)


def _deg_body(K, dst_hbm, out_hbm, dstv, onev, zv, idxv, buf, deg_sh, sem):
    c = lax.axis_index("c")
    s = lax.axis_index("s")
    wid = c * NS + s

    zero16 = jnp.zeros((16,), jnp.float32)
    onehot = jnp.where(lax.iota(jnp.int32, 16) == 0, 1.0, 0.0).astype(jnp.float32)

    def init_bufs(i, _):
        zv[i] = zero16
        onev[i] = onehot
        return _

    lax.fori_loop(0, CHUNK, init_bufs, None)
    _fill_slice_indices(idxv, s, CHUNK, G)

    for g in range(G):
        pltpu.sync_copy(zv, deg_sh.at[idxv.at[g]])
    plsc.subcore_barrier()

    def step(j, _):
        pltpu.sync_copy(dst_hbm.at[pl.ds(wid * K + j, 1)], dstv)
        pltpu.sync_copy(onev, deg_sh.at[dstv.at[0]], add=True)
        return _

    lax.fori_loop(0, K, step, None)
    plsc.subcore_barrier()

    for g in range(G):
        pltpu.async_copy(deg_sh.at[idxv.at[g]], buf, sem).wait()
        pltpu.sync_copy(buf, out_hbm.at[c, pl.ds(s * RPT + g * CHUNK, CHUNK)])


def _make_deg_kernel(K):
    return pl.kernel(
        functools.partial(_deg_body, K),
        out_type=jax.ShapeDtypeStruct((NC, N_PAD, 16), jnp.float32),
        mesh=_mesh(),
        scratch_types=[
            pltpu.VMEM((1, CHUNK), jnp.int32),
            pltpu.VMEM((CHUNK, 16), jnp.float32),
            pltpu.VMEM((CHUNK, 16), jnp.float32),
            pltpu.VMEM((G, CHUNK), jnp.int32),
            pltpu.VMEM((CHUNK, 16), jnp.float32),
            pltpu.VMEM_SHARED((N_PAD, 16), jnp.float32),
            pltpu.SemaphoreType.DMA,
        ],
    )


def _agg_body(
    K, y_hbm, src_hbm, dst_hbm, out_hbm, srcb, dstb, rows, idxv, acc_sh,
    sem_g, sem_s, sem_d, sem_r
):
    c = lax.axis_index("c")
    s = lax.axis_index("s")
    wid = c * NS + s

    zero16 = jnp.zeros((16,), jnp.float32)

    def zero_rows(k, _):
        rows[0][k // 8, pl.ds((k % 8) * 16, 16)] = zero16
        return _

    lax.fori_loop(0, ECHUNK * 8, zero_rows, None)
    _fill_slice_indices(idxv, s, ECHUNK, GA)

    for g in range(GA):
        pltpu.sync_copy(rows[0], acc_sh.at[idxv.at[g]])
    plsc.subcore_barrier()

    def load_src(j, b):
        pltpu.async_copy(src_hbm.at[pl.ds(wid * K + j, 1)], srcb[b], sem_r[b])

    def wait_src(b):
        pltpu.make_async_copy(src_hbm.at[pl.ds(0, 1)], srcb[b], sem_r[b]).wait()

    def load_dst(j, b):
        pltpu.async_copy(dst_hbm.at[pl.ds(wid * K + j, 1)], dstb[b], sem_d[b])

    def wait_dst(b):
        pltpu.make_async_copy(dst_hbm.at[pl.ds(0, 1)], dstb[b], sem_d[b]).wait()

    H = ECHUNK // 2

    def start_gather(b):
        pltpu.async_copy(
            y_hbm.at[srcb[b].at[0, pl.ds(0, H)]], rows[b].at[pl.ds(0, H)],
            sem_g[b],
        )
        pltpu.async_copy(
            y_hbm.at[srcb[b].at[0, pl.ds(H, H)]], rows[b].at[pl.ds(H, H)],
            sem_s[b],
        )

    def wait_gather(b):
        pltpu.make_async_copy(
            y_hbm.at[srcb[b].at[0, pl.ds(0, H)]], rows[b].at[pl.ds(0, H)],
            sem_g[b],
        ).wait()
        pltpu.make_async_copy(
            y_hbm.at[srcb[b].at[0, pl.ds(H, H)]], rows[b].at[pl.ds(H, H)],
            sem_s[b],
        ).wait()

    def scatter_sync(b):
        pltpu.sync_copy(rows[b], acc_sh.at[dstb[b].at[0]], add=True)

    pltpu.sync_copy(src_hbm.at[pl.ds(wid * K, 1)], srcb[0])
    pltpu.sync_copy(src_hbm.at[pl.ds(wid * K + 1, 1)], srcb[1])
    load_dst(0, 0)
    load_dst(1, 1)
    start_gather(0)

    def step(m, _):
        j0 = 2 * m
        wait_gather(0)
        start_gather(1)
        wait_dst(0)
        scatter_sync(0)

        @pl.when(j0 + 2 < K)
        def _next0():
            load_src(j0 + 2, 0)
            load_dst(j0 + 2, 0)

        wait_gather(1)

        @pl.when(j0 + 2 < K)
        def _start0():
            wait_src(0)
            start_gather(0)

        wait_dst(1)
        scatter_sync(1)

        @pl.when(j0 + 3 < K)
        def _next1():
            load_src(j0 + 3, 1)
            load_dst(j0 + 3, 1)
            wait_src(1)

        return _

    lax.fori_loop(0, K // 2, step, None)
    plsc.subcore_barrier()

    for g in range(GA):
        pltpu.async_copy(acc_sh.at[idxv.at[g]], rows[0], sem_g[0]).wait()
        pltpu.sync_copy(rows[0], out_hbm.at[c, pl.ds(s * RPT + g * ECHUNK, ECHUNK)])


def _make_agg_kernel(K):
    body = functools.partial(_agg_body, K)

    def wrapped(y_hbm, src_hbm, dst_hbm, out_hbm, *scratch):
        srcb = scratch[0:NBUF]
        dstb = scratch[NBUF:2 * NBUF]
        rows = scratch[2 * NBUF:3 * NBUF]
        idxv = scratch[3 * NBUF]
        acc_sh = scratch[3 * NBUF + 1]
        sems = scratch[3 * NBUF + 2:]
        sem_g = sems[0:NBUF]
        sem_s = sems[NBUF:2 * NBUF]
        sem_d = sems[2 * NBUF:3 * NBUF]
        sem_r = sems[3 * NBUF:4 * NBUF]
        body(y_hbm, src_hbm, dst_hbm, out_hbm, srcb, dstb, rows, idxv, acc_sh,
             sem_g, sem_s, sem_d, sem_r)

    return pl.kernel(
        wrapped,
        out_type=jax.ShapeDtypeStruct((NC, N_PAD, D), jnp.float32),
        mesh=_mesh(),
        scratch_types=[
            *[pltpu.VMEM((1, ECHUNK), jnp.int32) for _ in range(NBUF)],
            *[pltpu.VMEM((1, ECHUNK), jnp.int32) for _ in range(NBUF)],
            *[pltpu.VMEM((ECHUNK, D), jnp.float32) for _ in range(NBUF)],
            pltpu.VMEM((GA, ECHUNK), jnp.int32),
            pltpu.VMEM_SHARED((N_PAD, D), jnp.float32),
            *[pltpu.SemaphoreType.DMA for _ in range(4 * NBUF)],
        ],
    )


_BLK = 1280
_GRID = N_PAD // _BLK


def _dis_from(deg_ref):
    deg = jnp.sum(deg_ref[0] + deg_ref[1], axis=1) + 1.0
    return lax.rsqrt(deg)


def _y1_body(x_ref, w_ref, deg_ref, y_ref):
    dis = _dis_from(deg_ref)
    xw = jnp.dot(x_ref[...], w_ref[...], preferred_element_type=jnp.float32)
    y_ref[...] = xw * dis[:, None]


def _mid_body(acc_ref, y_ref, deg_ref, w_ref, b_ref, out_ref):
    dis = _dis_from(deg_ref)
    tot = acc_ref[0] + acc_ref[1] + y_ref[...]
    h = jnp.maximum(tot * dis[:, None] + b_ref[...], 0.0)
    out_ref[...] = jnp.dot(h, w_ref[...], preferred_element_type=jnp.float32) * dis[:, None]


def _fin_body(acc_ref, y_ref, deg_ref, b_ref, out_ref):
    dis = _dis_from(deg_ref)
    tot = acc_ref[0] + acc_ref[1] + y_ref[...]
    out_ref[...] = tot * dis[:, None] + b_ref[...]


_deg_spec = pl.BlockSpec((NC, _BLK, 16), lambda i: (0, i, 0))
_acc_spec = pl.BlockSpec((NC, _BLK, D), lambda i: (0, i, 0))
_row_spec = pl.BlockSpec((_BLK, D), lambda i: (i, 0))
_w_spec = pl.BlockSpec((D, D), lambda i: (0, 0))
_b_spec = pl.BlockSpec((1, D), lambda i: (0, 0))


def _tc_call(body, in_specs):
    return pl.pallas_call(
        body,
        grid=(_GRID,),
        in_specs=in_specs,
        out_specs=_row_spec,
        out_shape=jax.ShapeDtypeStruct((N_PAD, D), jnp.float32),
    )


def kernel(x, edge_index, W1, b1, W2, b2):
    E = edge_index.shape[1]
    K = -(-E // (NW * ECHUNK))
    K += (-K) % NBUF
    e_pad = NW * K * ECHUNK - E

    src = edge_index[0].astype(jnp.int32)
    dst = edge_index[1].astype(jnp.int32)
    padv = jnp.full((e_pad,), N_NODES, jnp.int32)
    src_p = jnp.concatenate([src, padv]).reshape(NW * K, ECHUNK)
    dst_p = jnp.concatenate([dst, padv]).reshape(NW * K, ECHUNK)
    kd = NW * K * ECHUNK // (NW * CHUNK)
    src_d = src_p.reshape(NW * kd, CHUNK)
    dst_d = dst_p.reshape(NW * kd, CHUNK)
    x_p = jnp.pad(x, ((0, N_PAD - N_NODES), (0, 0)))
    b1r = b1.reshape(1, D)
    b2r = b2.reshape(1, D)

    deg = _make_deg_kernel(kd)(dst_d)

    y1 = _tc_call(_y1_body, [_row_spec, _w_spec, _deg_spec])(x_p, W1, deg)
    acc1 = _make_agg_kernel(K)(y1, src_p, dst_p)
    y2 = _tc_call(_mid_body, [_acc_spec, _row_spec, _deg_spec, _w_spec, _b_spec])(
        acc1, y1, deg, W2, b1r
    )
    acc2 = _make_agg_kernel(K)(y2, src_p, dst_p)
    out = _tc_call(_fin_body, [_acc_spec, _row_spec, _deg_spec, _b_spec])(
        acc2, y2, deg, b2r
    )
    return out[:N_NODES]


# speedup vs baseline: 1.0265x; 1.0265x over previous
import functools

import jax
import jax.numpy as jnp
from jax import lax
from jax.experimental import pallas as pl
from jax.experimental.pallas import tpu as pltpu
from jax.experimental.pallas import tpu_sc as plsc

N_NODES = 10000
D = 128
NC = 2
NS = 16
NW = NC * NS
CHUNK = 128
ECHUNK = 128
NBUF = 2
N_PAD = 10240
RPT = N_PAD // NS
G = RPT // CHUNK
GA = RPT // ECHUNK


def _mesh():
    return plsc.VectorSubcoreMesh(
        core_axis_name="c", subcore_axis_name="s", num_cores=NC, num_subcores=NS
    )


def _fill_slice_indices(idxv, s, chunk, groups):
    iota16 = lax.iota(jnp.int32, 16)
    per_row = chunk // 16

    def fill(k, _):
        g = k // per_row
        t = k % per_row
        idxv[g, pl.ds(t * 16, 16)] = s * RPT + g * chunk + t * 16 + iota16
        return _

    lax.fori_loop(0, groups * per_row, fill, None)


def _deg_body(K, dst_hbm, out_hbm, dstv, onev, zv, idxv, buf, deg_sh, sem, semb):
    c = lax.axis_index("c")
    s = lax.axis_index("s")
    wid = c * NS + s

    zero16 = jnp.zeros((16,), jnp.float32)
    onehot = jnp.where(lax.iota(jnp.int32, 16) == 0, 1.0, 0.0).astype(jnp.float32)

    def init_bufs(i, _):
        zv[i] = zero16
        onev[i] = onehot
        return _

    lax.fori_loop(0, CHUNK, init_bufs, None)
    _fill_slice_indices(idxv, s, CHUNK, G)

    for g in range(G):
        pltpu.sync_copy(zv, deg_sh.at[idxv.at[g]])
    plsc.subcore_barrier()

    def load_idx(j, p, sem_p):
        pltpu.async_copy(
            dst_hbm.at[pl.ds(wid * K + j, 1)], dstv.at[pl.ds(p, 1)], sem_p
        )

    def wait_idx(p, sem_p):
        pltpu.make_async_copy(
            dst_hbm.at[pl.ds(0, 1)], dstv.at[pl.ds(p, 1)], sem_p
        ).wait()

    load_idx(0, 0, sem)
    load_idx(1, 1, semb)

    def step(m, _):
        j0 = 2 * m
        wait_idx(0, sem)
        pltpu.sync_copy(onev, deg_sh.at[dstv.at[0]], add=True)

        @pl.when(j0 + 2 < K)
        def _n0():
            load_idx(j0 + 2, 0, sem)

        wait_idx(1, semb)
        pltpu.sync_copy(onev, deg_sh.at[dstv.at[1]], add=True)

        @pl.when(j0 + 3 < K)
        def _n1():
            load_idx(j0 + 3, 1, semb)

        return _

    lax.fori_loop(0, K // 2, step, None)
    plsc.subcore_barrier()

    for g in range(G):
        pltpu.async_copy(deg_sh.at[idxv.at[g]], buf, sem).wait()
        pltpu.sync_copy(buf, out_hbm.at[c, pl.ds(s * RPT + g * CHUNK, CHUNK)])


def _make_deg_kernel(K):
    return pl.kernel(
        functools.partial(_deg_body, K),
        out_type=jax.ShapeDtypeStruct((NC, N_PAD, 16), jnp.float32),
        mesh=_mesh(),
        scratch_types=[
            pltpu.VMEM((2, CHUNK), jnp.int32),
            pltpu.VMEM((CHUNK, 16), jnp.float32),
            pltpu.VMEM((CHUNK, 16), jnp.float32),
            pltpu.VMEM((G, CHUNK), jnp.int32),
            pltpu.VMEM((CHUNK, 16), jnp.float32),
            pltpu.VMEM_SHARED((N_PAD, 16), jnp.float32),
            pltpu.SemaphoreType.DMA,
            pltpu.SemaphoreType.DMA,
        ],
    )


def _agg_body(
    K, y_hbm, src_hbm, dst_hbm, out_hbm, srcb, dstb, rows, idxv, acc_sh,
    sem_g, sem_s, sem_d, sem_r
):
    c = lax.axis_index("c")
    s = lax.axis_index("s")
    wid = c * NS + s

    zero16 = jnp.zeros((16,), jnp.float32)

    def zero_rows(k, _):
        rows[0][k // 8, pl.ds((k % 8) * 16, 16)] = zero16
        return _

    lax.fori_loop(0, ECHUNK * 8, zero_rows, None)
    _fill_slice_indices(idxv, s, ECHUNK, GA)

    for g in range(GA):
        pltpu.sync_copy(rows[0], acc_sh.at[idxv.at[g]])
    plsc.subcore_barrier()

    def load_src(j, b):
        pltpu.async_copy(src_hbm.at[pl.ds(wid * K + j, 1)], srcb[b], sem_r[b])

    def wait_src(b):
        pltpu.make_async_copy(src_hbm.at[pl.ds(0, 1)], srcb[b], sem_r[b]).wait()

    def load_dst(j, b):
        pltpu.async_copy(dst_hbm.at[pl.ds(wid * K + j, 1)], dstb[b], sem_d[b])

    def wait_dst(b):
        pltpu.make_async_copy(dst_hbm.at[pl.ds(0, 1)], dstb[b], sem_d[b]).wait()

    def start_gather(b):
        pltpu.async_copy(y_hbm.at[srcb[b].at[0]], rows[b], sem_g[b])

    def wait_gather(b):
        pltpu.make_async_copy(y_hbm.at[srcb[b].at[0]], rows[b], sem_g[b]).wait()

    def scatter_sync(b):
        pltpu.sync_copy(rows[b], acc_sh.at[dstb[b].at[0]], add=True)

    pltpu.sync_copy(src_hbm.at[pl.ds(wid * K, 1)], srcb[0])
    pltpu.sync_copy(src_hbm.at[pl.ds(wid * K + 1, 1)], srcb[1])
    load_dst(0, 0)
    load_dst(1, 1)
    start_gather(0)

    def step(m, _):
        j0 = 2 * m
        wait_gather(0)
        start_gather(1)
        wait_dst(0)
        scatter_sync(0)

        @pl.when(j0 + 2 < K)
        def _next0():
            load_src(j0 + 2, 0)
            load_dst(j0 + 2, 0)

        wait_gather(1)

        @pl.when(j0 + 2 < K)
        def _start0():
            wait_src(0)
            start_gather(0)

        wait_dst(1)
        scatter_sync(1)

        @pl.when(j0 + 3 < K)
        def _next1():
            load_src(j0 + 3, 1)
            load_dst(j0 + 3, 1)
            wait_src(1)

        return _

    lax.fori_loop(0, K // 2, step, None)
    plsc.subcore_barrier()

    for g in range(GA):
        pltpu.async_copy(acc_sh.at[idxv.at[g]], rows[0], sem_g[0]).wait()
        pltpu.sync_copy(rows[0], out_hbm.at[c, pl.ds(s * RPT + g * ECHUNK, ECHUNK)])


def _make_agg_kernel(K):
    body = functools.partial(_agg_body, K)

    def wrapped(y_hbm, src_hbm, dst_hbm, out_hbm, *scratch):
        srcb = scratch[0:NBUF]
        dstb = scratch[NBUF:2 * NBUF]
        rows = scratch[2 * NBUF:3 * NBUF]
        idxv = scratch[3 * NBUF]
        acc_sh = scratch[3 * NBUF + 1]
        sems = scratch[3 * NBUF + 2:]
        sem_g = sems[0:NBUF]
        sem_s = sems[NBUF:2 * NBUF]
        sem_d = sems[2 * NBUF:3 * NBUF]
        sem_r = sems[3 * NBUF:4 * NBUF]
        body(y_hbm, src_hbm, dst_hbm, out_hbm, srcb, dstb, rows, idxv, acc_sh,
             sem_g, sem_s, sem_d, sem_r)

    return pl.kernel(
        wrapped,
        out_type=jax.ShapeDtypeStruct((NC, N_PAD, D), jnp.float32),
        mesh=_mesh(),
        scratch_types=[
            *[pltpu.VMEM((1, ECHUNK), jnp.int32) for _ in range(NBUF)],
            *[pltpu.VMEM((1, ECHUNK), jnp.int32) for _ in range(NBUF)],
            *[pltpu.VMEM((ECHUNK, D), jnp.float32) for _ in range(NBUF)],
            pltpu.VMEM((GA, ECHUNK), jnp.int32),
            pltpu.VMEM_SHARED((N_PAD, D), jnp.float32),
            *[pltpu.SemaphoreType.DMA for _ in range(4 * NBUF)],
        ],
    )


_BLK = 1280
_GRID = N_PAD // _BLK


def _dis_from(deg_ref):
    deg = jnp.sum(deg_ref[0] + deg_ref[1], axis=1) + 1.0
    return lax.rsqrt(deg)


def _y1_body(x_ref, w_ref, deg_ref, y_ref):
    dis = _dis_from(deg_ref)
    xw = jnp.dot(x_ref[...], w_ref[...], preferred_element_type=jnp.float32)
    y_ref[...] = xw * dis[:, None]


def _mid_body(acc_ref, y_ref, deg_ref, w_ref, b_ref, out_ref):
    dis = _dis_from(deg_ref)
    tot = acc_ref[0] + acc_ref[1] + y_ref[...]
    h = jnp.maximum(tot * dis[:, None] + b_ref[...], 0.0)
    out_ref[...] = jnp.dot(h, w_ref[...], preferred_element_type=jnp.float32) * dis[:, None]


def _fin_body(acc_ref, y_ref, deg_ref, b_ref, out_ref):
    dis = _dis_from(deg_ref)
    tot = acc_ref[0] + acc_ref[1] + y_ref[...]
    out_ref[...] = tot * dis[:, None] + b_ref[...]


_deg_spec = pl.BlockSpec((NC, _BLK, 16), lambda i: (0, i, 0))
_acc_spec = pl.BlockSpec((NC, _BLK, D), lambda i: (0, i, 0))
_row_spec = pl.BlockSpec((_BLK, D), lambda i: (i, 0))
_w_spec = pl.BlockSpec((D, D), lambda i: (0, 0))
_b_spec = pl.BlockSpec((1, D), lambda i: (0, 0))


def _tc_call(body, in_specs):
    return pl.pallas_call(
        body,
        grid=(_GRID,),
        in_specs=in_specs,
        out_specs=_row_spec,
        out_shape=jax.ShapeDtypeStruct((N_PAD, D), jnp.float32),
    )


def kernel(x, edge_index, W1, b1, W2, b2):
    E = edge_index.shape[1]
    K = -(-E // (NW * ECHUNK))
    K += (-K) % NBUF
    e_pad = NW * K * ECHUNK - E

    src = edge_index[0].astype(jnp.int32)
    dst = edge_index[1].astype(jnp.int32)
    padv = jnp.full((e_pad,), N_NODES, jnp.int32)
    src_p = jnp.concatenate([src, padv]).reshape(NW * K, ECHUNK)
    dst_p = jnp.concatenate([dst, padv]).reshape(NW * K, ECHUNK)
    kd = NW * K * ECHUNK // (NW * CHUNK)
    src_d = src_p.reshape(NW * kd, CHUNK)
    dst_d = dst_p.reshape(NW * kd, CHUNK)
    x_p = jnp.pad(x, ((0, N_PAD - N_NODES), (0, 0)))
    b1r = b1.reshape(1, D)
    b2r = b2.reshape(1, D)

    deg = _make_deg_kernel(kd)(dst_d)

    y1 = _tc_call(_y1_body, [_row_spec, _w_spec, _deg_spec])(x_p, W1, deg)
    acc1 = _make_agg_kernel(K)(y1, src_p, dst_p)
    y2 = _tc_call(_mid_body, [_acc_spec, _row_spec, _deg_spec, _w_spec, _b_spec])(
        acc1, y1, deg, W2, b1r
    )
    acc2 = _make_agg_kernel(K)(y2, src_p, dst_p)
    out = _tc_call(_fin_body, [_acc_spec, _row_spec, _deg_spec, _b_spec])(
        acc2, y2, deg, b2r
    )
    return out[:N_NODES]
